# Initial kernel scaffold; baseline (speedup 1.0000x reference)
import jax
import jax.numpy as jnp
from jax.experimental import pallas as pl


def kernel(x, W, b, codebook):
    raise NotImplementedError("write your pallas kernel here")


# speedup vs baseline: 16.4903x; 16.4903x over previous
import jax
import jax.numpy as jnp
from jax.experimental import pallas as pl
from jax.experimental.pallas import tpu as pltpu

_NUM_CENTROIDS = 8
_LOSS_SCALE = 1.25


def _vq_block_kernel(x_ref, w_ref, b_ref, cb_ref,
                     q_ref, idx_ref, tw_ref, enc_ref, loss_ref):
    x = x_ref[...]
    logits = jnp.dot(x, w_ref[...],
                     preferred_element_type=jnp.float32) + b_ref[...]
    k = logits.shape[1]
    iota = jax.lax.broadcasted_iota(jnp.int32, logits.shape, 1)

    vals = logits
    top_v, top_i = [], []
    for _ in range(_NUM_CENTROIDS):
        m = jnp.max(vals, axis=1, keepdims=True)
        am = jnp.min(jnp.where(vals == m, iota, k), axis=1, keepdims=True)
        top_v.append(m)
        top_i.append(am)
        vals = jnp.where(iota == am, -jnp.inf, vals)
    tv = jnp.concatenate(top_v, axis=1)
    ti = jnp.concatenate(top_i, axis=1)

    e = jnp.exp(tv - tv[:, :1])
    tw = e / jnp.sum(e, axis=1, keepdims=True)

    idx_ref[...] = ti
    tw_ref[...] = tw

    enc = jnp.zeros_like(logits)
    for h in range(_NUM_CENTROIDS):
        enc += jnp.where(iota == ti[:, h:h + 1], tw[:, h:h + 1], 0.0)
    enc_ref[...] = enc

    q = jnp.dot(enc, cb_ref[...], preferred_element_type=jnp.float32)
    q_ref[...] = q

    part = jnp.sum((q - x) ** 2)

    @pl.when(pl.program_id(0) == 0)
    def _init():
        loss_ref[...] = jnp.zeros_like(loss_ref)

    loss_ref[...] += part.reshape(1, 1)


def kernel(x, W, b, codebook):
    B, T, D = x.shape
    K = codebook.shape[0]
    N = B * T
    BLK = 512
    grid = N // BLK
    xf = x.reshape(N, D)

    q, ti, tw, enc, loss = pl.pallas_call(
        _vq_block_kernel,
        grid=(grid,),
        in_specs=[
            pl.BlockSpec((BLK, D), lambda i: (i, 0)),
            pl.BlockSpec((D, K), lambda i: (0, 0)),
            pl.BlockSpec((K,), lambda i: (0,)),
            pl.BlockSpec((K, D), lambda i: (0, 0)),
        ],
        out_specs=[
            pl.BlockSpec((BLK, D), lambda i: (i, 0)),
            pl.BlockSpec((BLK, _NUM_CENTROIDS), lambda i: (i, 0)),
            pl.BlockSpec((BLK, _NUM_CENTROIDS), lambda i: (i, 0)),
            pl.BlockSpec((BLK, K), lambda i: (i, 0)),
            pl.BlockSpec((1, 1), lambda i: (0, 0)),
        ],
        out_shape=[
            jax.ShapeDtypeStruct((N, D), jnp.float32),
            jax.ShapeDtypeStruct((N, _NUM_CENTROIDS), jnp.int32),
            jax.ShapeDtypeStruct((N, _NUM_CENTROIDS), jnp.float32),
            jax.ShapeDtypeStruct((N, K), jnp.float32),
            jax.ShapeDtypeStruct((1, 1), jnp.float32),
        ],
        compiler_params=pltpu.CompilerParams(
            dimension_semantics=("arbitrary",),
        ),
    )(xf, W, b, codebook)

    quantized_st = q.reshape(B, T, D)
    top_indices = ti.reshape(B, T, _NUM_CENTROIDS)
    top_weights = tw.reshape(B, T, _NUM_CENTROIDS)
    encodings = enc.reshape(B, T, K)
    loss_out = (loss[0, 0] * _LOSS_SCALE) / (N * D)
    return (quantized_st, top_indices, top_weights, encodings, loss_out)


# speedup vs baseline: 19.0928x; 1.1578x over previous
import jax
import jax.numpy as jnp
from jax.experimental import pallas as pl
from jax.experimental.pallas import tpu as pltpu

_NUM_CENTROIDS = 8
_LOSS_SCALE = 1.25


def _vq_block_kernel(x_ref, w_ref, b_ref, cb_ref,
                     q_ref, idx_ref, tw_ref, enc_ref, loss_ref):
    x = x_ref[...]
    logits = jnp.dot(x, w_ref[...],
                     preferred_element_type=jnp.float32) + b_ref[...]
    k = logits.shape[1]
    iota_f = jax.lax.broadcasted_iota(
        jnp.int32, logits.shape, 1).astype(jnp.float32)
    kf = jnp.float32(k)

    vals = logits
    top_v, top_i = [], []
    for _ in range(_NUM_CENTROIDS):
        m = jnp.max(vals, axis=1, keepdims=True)
        am = jnp.min(jnp.where(vals == m, iota_f, kf), axis=1, keepdims=True)
        top_v.append(m)
        top_i.append(am)
        vals = jnp.where(iota_f == am, -jnp.inf, vals)
    tv = jnp.concatenate(top_v, axis=1)
    tif = jnp.concatenate(top_i, axis=1)

    e = jnp.exp(tv - tv[:, :1])
    tw = e / jnp.sum(e, axis=1, keepdims=True)

    idx_ref[...] = tif.astype(jnp.int32)
    tw_ref[...] = tw

    enc = jnp.zeros_like(logits)
    for h in range(_NUM_CENTROIDS):
        enc += jnp.where(iota_f == tif[:, h:h + 1], tw[:, h:h + 1], 0.0)
    enc_ref[...] = enc

    q = jnp.dot(enc, cb_ref[...], preferred_element_type=jnp.float32)
    q_ref[...] = q

    part = jnp.sum((q - x) ** 2)

    @pl.when(pl.program_id(0) == 0)
    def _init():
        loss_ref[...] = jnp.zeros_like(loss_ref)

    loss_ref[...] += part.reshape(1, 1)


def kernel(x, W, b, codebook):
    B, T, D = x.shape
    K = codebook.shape[0]
    N = B * T
    BLK = 512
    grid = N // BLK
    xf = x.reshape(N, D)

    q, ti, tw, enc, loss = pl.pallas_call(
        _vq_block_kernel,
        grid=(grid,),
        in_specs=[
            pl.BlockSpec((BLK, D), lambda i: (i, 0)),
            pl.BlockSpec((D, K), lambda i: (0, 0)),
            pl.BlockSpec((K,), lambda i: (0,)),
            pl.BlockSpec((K, D), lambda i: (0, 0)),
        ],
        out_specs=[
            pl.BlockSpec((BLK, D), lambda i: (i, 0)),
            pl.BlockSpec((BLK, _NUM_CENTROIDS), lambda i: (i, 0)),
            pl.BlockSpec((BLK, _NUM_CENTROIDS), lambda i: (i, 0)),
            pl.BlockSpec((BLK, K), lambda i: (i, 0)),
            pl.BlockSpec((1, 1), lambda i: (0, 0)),
        ],
        out_shape=[
            jax.ShapeDtypeStruct((N, D), jnp.float32),
            jax.ShapeDtypeStruct((N, _NUM_CENTROIDS), jnp.int32),
            jax.ShapeDtypeStruct((N, _NUM_CENTROIDS), jnp.float32),
            jax.ShapeDtypeStruct((N, K), jnp.float32),
            jax.ShapeDtypeStruct((1, 1), jnp.float32),
        ],
        compiler_params=pltpu.CompilerParams(
            dimension_semantics=("arbitrary",),
        ),
    )(xf, W, b, codebook)

    quantized_st = q.reshape(B, T, D)
    top_indices = ti.reshape(B, T, _NUM_CENTROIDS)
    top_weights = tw.reshape(B, T, _NUM_CENTROIDS)
    encodings = enc.reshape(B, T, K)
    loss_out = (loss[0, 0] * _LOSS_SCALE) / (N * D)
    return (quantized_st, top_indices, top_weights, encodings, loss_out)
